# baseline (speedup 1.0000x reference)
import functools

import jax
import jax.numpy as jnp
from jax import lax
from jax.experimental import pallas as pl
from jax.experimental.pallas import tpu as pltpu
from jax.experimental.pallas import tpu_sc as plsc

_NUM_TYPES = 64
_N = 100000
_LANES = 16
_NC, _NS = 2, 16
_NW = _NC * _NS
_CW = 128
_FULL_CHUNKS = _N // _CW
_TAIL = _N - _FULL_CHUNKS * _CW
_SLOTS = -(-_FULL_CHUNKS // _NW)
_GP = _CW // _LANES
_TAIL_GP = _TAIL // _LANES
_TAIL_W = _NW - 1


def _one_hot_t_sc(types_flat):
    mesh = plsc.VectorSubcoreMesh(
        core_axis_name="c", subcore_axis_name="s",
        num_cores=_NC, num_subcores=_NS,
    )

    @functools.partial(
        pl.kernel,
        mesh=mesh,
        out_type=jax.ShapeDtypeStruct((_NUM_TYPES, _N), jnp.float32),
        scratch_types=[
            pltpu.VMEM((_SLOTS * _CW,), jnp.int32),
            pltpu.VMEM((_TAIL,), jnp.int32),
            pltpu.VMEM((_NUM_TYPES, _CW), jnp.float32),
            pltpu.VMEM((_NUM_TYPES, _CW), jnp.float32),
            pltpu.VMEM((_NUM_TYPES, _TAIL), jnp.float32),
            pltpu.SemaphoreType.DMA,
            pltpu.SemaphoreType.DMA,
            pltpu.SemaphoreType.DMA,
            pltpu.SemaphoreType.DMA,
        ],
        compiler_params=pltpu.CompilerParams(needs_layout_passes=False,
                                             use_tc_tiling_on_sc=True),
    )
    def k(types_hbm, out_a_hbm,
          idx_all, idx_tail, r0, r1, r_tail,
          in_sem, osem0, osem1, tsem):
        rows = (r0, r1)
        osem = (osem0, osem1)

        wid = lax.axis_index("s") * _NC + lax.axis_index("c")

        def chunk_col(s):
            k_s = wid + _NW * s
            k_s = jnp.where(k_s < _FULL_CHUNKS, k_s, wid)
            return pl.multiple_of(k_s * _CW, _CW)

        def stage_body(s, carry):
            pltpu.make_async_copy(
                types_hbm.at[pl.ds(chunk_col(s), _CW)],
                idx_all.at[pl.ds(s * _CW, _CW)], in_sem).start()
            return carry

        lax.fori_loop(0, _SLOTS, stage_body, 0)
        tail_in = pltpu.make_async_copy(
            types_hbm.at[pl.ds(_FULL_CHUNKS * _CW, _TAIL)], idx_tail,
            in_sem)
        tail_in.start()

        lane = lax.iota(jnp.int32, _LANES)
        ones = jnp.full((_LANES,), 1.0, jnp.float32)
        zeros = jnp.zeros((_LANES,), jnp.float32)

        def zbody(i, carry):
            rvec = jnp.full((_LANES,), 0, jnp.int32) + i
            for u in range(_CW // _LANES):
                cvec = lane + u * _LANES
                plsc.store_scatter(r0, [rvec, cvec], zeros)
                plsc.store_scatter(r1, [rvec, cvec], zeros)
            for u in range(_TAIL // _LANES):
                cvec = lane + u * _LANES
                plsc.store_scatter(r_tail, [rvec, cvec], zeros)
            return carry

        lax.fori_loop(0, _NUM_TYPES, zbody, 0, unroll=4)

        def drain_in(s, carry):
            pltpu.make_async_copy(
                types_hbm.at[pl.ds(0, _CW)],
                idx_all.at[pl.ds(0, _CW)], in_sem).wait()
            return carry

        lax.fori_loop(0, _SLOTS, drain_in, 0)
        tail_in.wait()

        def scatter_pass(row_ref, idx_base, ngroups, val):
            for j in range(ngroups):
                t = idx_all[pl.ds(idx_base + j * _LANES, _LANES)]
                plsc.store_scatter(row_ref, [t, lane + j * _LANES], val)

        def slot_body(s, b, first):
            col = chunk_col(s)
            if not first:
                pltpu.make_async_copy(
                    rows[b], out_a_hbm.at[:, pl.ds(col, _CW)],
                    osem[b]).wait()
                scatter_pass(rows[b], (s - 2) * _CW, _GP, zeros)
            scatter_pass(rows[b], s * _CW, _GP, ones)
            pltpu.make_async_copy(
                rows[b], out_a_hbm.at[:, pl.ds(col, _CW)], osem[b]).start()

        slot_body(0, 0, True)
        slot_body(1, 1, True)

        def pair_body(i, carry):
            s = 2 + 2 * i
            slot_body(s, 0, False)
            slot_body(s + 1, 1, False)
            return carry

        lax.fori_loop(0, (_SLOTS - 3) // 2, pair_body, 0)

        slot_body(_SLOTS - 1, 0, False)

        @pl.when(wid == _TAIL_W)
        def _():
            for j in range(_TAIL_GP):
                t = idx_tail[pl.ds(j * _LANES, _LANES)]
                plsc.store_scatter(r_tail, [t, lane + j * _LANES], ones)
            descs = [
                pltpu.make_async_copy(
                    r_tail,
                    out_a_hbm.at[:, pl.ds(_FULL_CHUNKS * _CW, _TAIL)],
                    tsem),
            ]
            for d in descs:
                d.start()
            for d in descs:
                d.wait()

        for s in (_SLOTS - 2, _SLOTS - 1):
            b = s & 1
            col = chunk_col(s)
            pltpu.make_async_copy(
                rows[b], out_a_hbm.at[:, pl.ds(col, _CW)], osem[b]).wait()

    return k(types_flat)


_TC_B = 2048


def _tc_body(t_ref, out_ref):
    t = t_ref[:]
    row = lax.broadcasted_iota(jnp.int32, (_NUM_TYPES, _TC_B), 0)
    out_ref[:, :] = (row == t[None, :]).astype(jnp.float32)


def _one_hot_t_tc(types_flat):
    grid = (-(-_N // _TC_B),)
    return pl.pallas_call(
        _tc_body,
        grid=grid,
        in_specs=[pl.BlockSpec((_TC_B,), lambda i: (i,))],
        out_specs=pl.BlockSpec((_NUM_TYPES, _TC_B), lambda i: (0, i)),
        out_shape=jax.ShapeDtypeStruct((_NUM_TYPES, _N), jnp.float32),
    )(types_flat)


def kernel(atom_types, pos):
    types_flat = atom_types.reshape(-1)
    out_a_t = _one_hot_t_sc(types_flat)
    out_b_t = _one_hot_t_tc(types_flat)
    return out_a_t.T.astype(pos.dtype), out_b_t.T.astype(pos.dtype)


# speedup vs baseline: 1.3470x; 1.3470x over previous
import functools

import jax
import jax.numpy as jnp
from jax import lax
from jax.experimental import pallas as pl
from jax.experimental.pallas import tpu as pltpu
from jax.experimental.pallas import tpu_sc as plsc

_NUM_TYPES = 64
_N = 100000
_LANES = 16
_NC, _NS = 2, 16
_NW = _NC * _NS
_CW = 128
_FULL_CHUNKS = _N // _CW
_TAIL = _N - _FULL_CHUNKS * _CW
_SLOTS = -(-_FULL_CHUNKS // _NW)
_GP = _CW // _LANES
_TAIL_GP = _TAIL // _LANES
_TAIL_W = _NW - 1


def _one_hot_t_sc(types_flat):
    mesh = plsc.VectorSubcoreMesh(
        core_axis_name="c", subcore_axis_name="s",
        num_cores=_NC, num_subcores=_NS,
    )

    @functools.partial(
        pl.kernel,
        mesh=mesh,
        out_type=jax.ShapeDtypeStruct((_NUM_TYPES, _N), jnp.float32),
        scratch_types=[
            pltpu.VMEM((_SLOTS * _CW,), jnp.int32),
            pltpu.VMEM((_TAIL,), jnp.int32),
            pltpu.VMEM((_NUM_TYPES, _CW), jnp.float32),
            pltpu.VMEM((_NUM_TYPES, _CW), jnp.float32),
            pltpu.VMEM((_NUM_TYPES, _TAIL), jnp.float32),
            pltpu.SemaphoreType.DMA,
            pltpu.SemaphoreType.DMA,
            pltpu.SemaphoreType.DMA,
            pltpu.SemaphoreType.DMA,
        ],
        compiler_params=pltpu.CompilerParams(needs_layout_passes=False,
                                             use_tc_tiling_on_sc=True),
    )
    def k(types_hbm, out_a_hbm,
          idx_all, idx_tail, r0, r1, r_tail,
          in_sem, osem0, osem1, tsem):
        rows = (r0, r1)
        osem = (osem0, osem1)

        wid = lax.axis_index("s") * _NC + lax.axis_index("c")

        def chunk_col(s):
            k_s = wid + _NW * s
            k_s = jnp.where(k_s < _FULL_CHUNKS, k_s, wid)
            return pl.multiple_of(k_s * _CW, _CW)

        def stage_body(s, carry):
            pltpu.make_async_copy(
                types_hbm.at[pl.ds(chunk_col(s), _CW)],
                idx_all.at[pl.ds(s * _CW, _CW)], in_sem).start()
            return carry

        lax.fori_loop(0, _SLOTS, stage_body, 0)
        tail_in = pltpu.make_async_copy(
            types_hbm.at[pl.ds(_FULL_CHUNKS * _CW, _TAIL)], idx_tail,
            in_sem)
        tail_in.start()

        lane = lax.iota(jnp.int32, _LANES)
        ones = jnp.full((_LANES,), 1.0, jnp.float32)
        zeros = jnp.zeros((_LANES,), jnp.float32)

        def zbody(i, carry):
            rvec = jnp.full((_LANES,), 0, jnp.int32) + i
            for u in range(_CW // _LANES):
                cvec = lane + u * _LANES
                plsc.store_scatter(r0, [rvec, cvec], zeros)
                plsc.store_scatter(r1, [rvec, cvec], zeros)
            for u in range(_TAIL // _LANES):
                cvec = lane + u * _LANES
                plsc.store_scatter(r_tail, [rvec, cvec], zeros)
            return carry

        lax.fori_loop(0, _NUM_TYPES, zbody, 0, unroll=4)

        def drain_in(s, carry):
            pltpu.make_async_copy(
                types_hbm.at[pl.ds(0, _CW)],
                idx_all.at[pl.ds(0, _CW)], in_sem).wait()
            return carry

        lax.fori_loop(0, _SLOTS, drain_in, 0)
        tail_in.wait()

        def scatter_pass(row_ref, idx_base, ngroups, val):
            for j in range(ngroups):
                t = idx_all[pl.ds(idx_base + j * _LANES, _LANES)]
                plsc.store_scatter(row_ref, [t, lane + j * _LANES], val)

        def slot_body(s, b, first):
            col = chunk_col(s)
            if not first:
                pltpu.make_async_copy(
                    rows[b], out_a_hbm.at[:, pl.ds(col, _CW)],
                    osem[b]).wait()
                scatter_pass(rows[b], (s - 2) * _CW, _GP, zeros)
            scatter_pass(rows[b], s * _CW, _GP, ones)
            pltpu.make_async_copy(
                rows[b], out_a_hbm.at[:, pl.ds(col, _CW)], osem[b]).start()

        slot_body(0, 0, True)
        slot_body(1, 1, True)

        def pair_body(i, carry):
            s = 2 + 2 * i
            slot_body(s, 0, False)
            slot_body(s + 1, 1, False)
            return carry

        lax.fori_loop(0, (_SLOTS - 3) // 2, pair_body, 0)

        slot_body(_SLOTS - 1, 0, False)

        @pl.when(wid == _TAIL_W)
        def _():
            for j in range(_TAIL_GP):
                t = idx_tail[pl.ds(j * _LANES, _LANES)]
                plsc.store_scatter(r_tail, [t, lane + j * _LANES], ones)
            descs = [
                pltpu.make_async_copy(
                    r_tail,
                    out_a_hbm.at[:, pl.ds(_FULL_CHUNKS * _CW, _TAIL)],
                    tsem),
            ]
            for d in descs:
                d.start()
            for d in descs:
                d.wait()

        for s in (_SLOTS - 2, _SLOTS - 1):
            b = s & 1
            col = chunk_col(s)
            pltpu.make_async_copy(
                rows[b], out_a_hbm.at[:, pl.ds(col, _CW)], osem[b]).wait()

    return k(types_flat)


_TC_B = 12288


def _tc_body(t_ref, out_ref):
    t = t_ref[:]
    row = lax.broadcasted_iota(jnp.int32, (_NUM_TYPES, _TC_B), 0)
    out_ref[:, :] = (row == t[None, :]).astype(jnp.float32)


def _one_hot_t_tc(types_flat):
    grid = (-(-_N // _TC_B),)
    return pl.pallas_call(
        _tc_body,
        grid=grid,
        in_specs=[pl.BlockSpec((_TC_B,), lambda i: (i,))],
        out_specs=pl.BlockSpec((_NUM_TYPES, _TC_B), lambda i: (0, i)),
        out_shape=jax.ShapeDtypeStruct((_NUM_TYPES, _N), jnp.float32),
    )(types_flat)


def kernel(atom_types, pos):
    types_flat = atom_types.reshape(-1)
    out_a_t = _one_hot_t_sc(types_flat)
    out_b_t = _one_hot_t_tc(types_flat)
    return out_a_t.T.astype(pos.dtype), out_b_t.T.astype(pos.dtype)


# speedup vs baseline: 1.3545x; 1.0056x over previous
import functools

import jax
import jax.numpy as jnp
from jax import lax
from jax.experimental import pallas as pl
from jax.experimental.pallas import tpu as pltpu
from jax.experimental.pallas import tpu_sc as plsc

_NUM_TYPES = 64
_N = 100000
_LANES = 16
_NC, _NS = 2, 16
_NW = _NC * _NS
_CW = 128
_FULL_CHUNKS = _N // _CW
_TAIL = _N - _FULL_CHUNKS * _CW
_SLOTS = -(-_FULL_CHUNKS // _NW)
_GP = _CW // _LANES
_TAIL_GP = _TAIL // _LANES
_TAIL_W = _NW - 1


def _one_hot_t_sc(types_flat):
    mesh = plsc.VectorSubcoreMesh(
        core_axis_name="c", subcore_axis_name="s",
        num_cores=_NC, num_subcores=_NS,
    )

    @functools.partial(
        pl.kernel,
        mesh=mesh,
        out_type=jax.ShapeDtypeStruct((_NUM_TYPES, _N), jnp.float32),
        scratch_types=[
            pltpu.VMEM((_SLOTS * _CW,), jnp.int32),
            pltpu.VMEM((_TAIL,), jnp.int32),
            pltpu.VMEM((_NUM_TYPES, _CW), jnp.float32),
            pltpu.VMEM((_NUM_TYPES, _CW), jnp.float32),
            pltpu.VMEM((_NUM_TYPES, _CW), jnp.float32),
            pltpu.VMEM((_NUM_TYPES, _CW), jnp.float32),
            pltpu.VMEM((_NUM_TYPES, _TAIL), jnp.float32),
            pltpu.SemaphoreType.DMA,
            pltpu.SemaphoreType.DMA,
            pltpu.SemaphoreType.DMA,
            pltpu.SemaphoreType.DMA,
            pltpu.SemaphoreType.DMA,
            pltpu.SemaphoreType.DMA,
        ],
        compiler_params=pltpu.CompilerParams(needs_layout_passes=False,
                                             use_tc_tiling_on_sc=True),
    )
    def k(types_hbm, out_a_hbm,
          idx_all, idx_tail, r0, r1, r2, r3, r_tail,
          in_sem, osem0, osem1, osem2, osem3, tsem):
        rows = (r0, r1, r2, r3)
        osem = (osem0, osem1, osem2, osem3)

        wid = lax.axis_index("s") * _NC + lax.axis_index("c")

        def chunk_col(s):
            k_s = wid + _NW * s
            k_s = jnp.where(k_s < _FULL_CHUNKS, k_s, wid)
            return pl.multiple_of(k_s * _CW, _CW)

        def stage_body(s, carry):
            pltpu.make_async_copy(
                types_hbm.at[pl.ds(chunk_col(s), _CW)],
                idx_all.at[pl.ds(s * _CW, _CW)], in_sem).start()
            return carry

        lax.fori_loop(0, _SLOTS, stage_body, 0)
        tail_in = pltpu.make_async_copy(
            types_hbm.at[pl.ds(_FULL_CHUNKS * _CW, _TAIL)], idx_tail,
            in_sem)
        tail_in.start()

        lane = lax.iota(jnp.int32, _LANES)
        ones = jnp.full((_LANES,), 1.0, jnp.float32)
        zeros = jnp.zeros((_LANES,), jnp.float32)

        def zbody(i, carry):
            rvec = jnp.full((_LANES,), 0, jnp.int32) + i
            for u in range(_CW // _LANES):
                cvec = lane + u * _LANES
                plsc.store_scatter(r0, [rvec, cvec], zeros)
                plsc.store_scatter(r1, [rvec, cvec], zeros)
                plsc.store_scatter(r2, [rvec, cvec], zeros)
                plsc.store_scatter(r3, [rvec, cvec], zeros)
            for u in range(_TAIL // _LANES):
                cvec = lane + u * _LANES
                plsc.store_scatter(r_tail, [rvec, cvec], zeros)
            return carry

        lax.fori_loop(0, _NUM_TYPES, zbody, 0, unroll=4)

        def drain_in(s, carry):
            pltpu.make_async_copy(
                types_hbm.at[pl.ds(0, _CW)],
                idx_all.at[pl.ds(0, _CW)], in_sem).wait()
            return carry

        lax.fori_loop(0, _SLOTS, drain_in, 0)
        tail_in.wait()

        def scatter_pass(row_ref, idx_base, ngroups, val):
            for j in range(ngroups):
                t = idx_all[pl.ds(idx_base + j * _LANES, _LANES)]
                plsc.store_scatter(row_ref, [t, lane + j * _LANES], val)

        def slot_body(s, b, first):
            col = chunk_col(s)
            if not first:
                pltpu.make_async_copy(
                    rows[b], out_a_hbm.at[:, pl.ds(col, _CW)],
                    osem[b]).wait()
                scatter_pass(rows[b], (s - 4) * _CW, _GP, zeros)
            scatter_pass(rows[b], s * _CW, _GP, ones)
            pltpu.make_async_copy(
                rows[b], out_a_hbm.at[:, pl.ds(col, _CW)], osem[b]).start()

        for s0 in range(4):
            slot_body(s0, s0, True)

        def quad_body(i, carry):
            s = 4 + 4 * i
            for q in range(4):
                slot_body(s + q, q, False)
            return carry

        lax.fori_loop(0, (_SLOTS - 5) // 4, quad_body, 0)

        slot_body(_SLOTS - 1, 0, False)

        @pl.when(wid == _TAIL_W)
        def _():
            for j in range(_TAIL_GP):
                t = idx_tail[pl.ds(j * _LANES, _LANES)]
                plsc.store_scatter(r_tail, [t, lane + j * _LANES], ones)
            descs = [
                pltpu.make_async_copy(
                    r_tail,
                    out_a_hbm.at[:, pl.ds(_FULL_CHUNKS * _CW, _TAIL)],
                    tsem),
            ]
            for d in descs:
                d.start()
            for d in descs:
                d.wait()

        for s in range(_SLOTS - 4, _SLOTS):
            b = s % 4
            col = chunk_col(s)
            pltpu.make_async_copy(
                rows[b], out_a_hbm.at[:, pl.ds(col, _CW)], osem[b]).wait()

    return k(types_flat)


_TC_B = 12288


def _tc_body(t_ref, out_ref):
    t = t_ref[:]
    row = lax.broadcasted_iota(jnp.int32, (_NUM_TYPES, _TC_B), 0)
    out_ref[:, :] = (row == t[None, :]).astype(jnp.float32)


def _one_hot_t_tc(types_flat):
    grid = (-(-_N // _TC_B),)
    return pl.pallas_call(
        _tc_body,
        grid=grid,
        in_specs=[pl.BlockSpec((_TC_B,), lambda i: (i,))],
        out_specs=pl.BlockSpec((_NUM_TYPES, _TC_B), lambda i: (0, i)),
        out_shape=jax.ShapeDtypeStruct((_NUM_TYPES, _N), jnp.float32),
    )(types_flat)


def kernel(atom_types, pos):
    types_flat = atom_types.reshape(-1)
    out_a_t = _one_hot_t_sc(types_flat)
    out_b_t = _one_hot_t_tc(types_flat)
    return out_a_t.T.astype(pos.dtype), out_b_t.T.astype(pos.dtype)
